# baseline (speedup 1.0000x reference)
import functools

import jax
import jax.numpy as jnp
from jax import lax
from jax.experimental import pallas as pl
from jax.experimental.pallas import tpu as pltpu
from jax.experimental.pallas import tpu_sc as plsc

NUM_POS = 28 * 28
EMB = 768
XD = 1536
BATCH = 64

HALF_POS = NUM_POS // 2
B_PER_W = 4
CHUNK = 8
NCH = HALF_POS // CHUNK
NBUF = 8
LOOK = 4
TSLOTS = 2
NITEM = NCH * B_PER_W
LANES = 16
NVEC = EMB // LANES
NROUND = (NCH - 1) // TSLOTS


def _sc_body(x_hbm, table_hbm, out_hbm, *refs):
    xbufs = refs[0:NBUF]
    tbufs = refs[NBUF:NBUF + TSLOTS]
    tshared = refs[NBUF + TSLOTS]
    lsems = refs[NBUF + TSLOTS + 1:2 * NBUF + TSLOTS + 1]
    ssems = refs[2 * NBUF + TSLOTS + 1:3 * NBUF + TSLOTS + 1]
    tsems = refs[3 * NBUF + TSLOTS + 1:3 * NBUF + 2 * TSLOTS + 1]
    core = lax.axis_index("c")
    sub = lax.axis_index("s")
    pbase = core * HALF_POS
    b0 = sub * B_PER_W

    @pl.when(sub == 0)
    def _():
        pltpu.sync_copy(table_hbm.at[pl.ds(pbase, HALF_POS)], tshared)
    plsc.subcore_barrier()

    def item_ci_nb(it):
        return it // B_PER_W, it % B_PER_W

    def load_desc(slot, it):
        ci, nb = item_ci_nb(it)
        p0 = pbase + ci * CHUNK
        return (pltpu.make_async_copy(
            x_hbm.at[b0 + nb, pl.ds(p0, CHUNK)],
            xbufs[slot], lsems[slot]),)

    def store_desc(slot, it):
        ci, nb = item_ci_nb(it)
        p0 = pbase + ci * CHUNK
        return (pltpu.make_async_copy(
            xbufs[slot],
            out_hbm.at[b0 + nb, pl.ds(p0, CHUNK)], ssems[slot]),)

    def tload_desc(ci, tslot):
        return (pltpu.make_async_copy(tshared.at[pl.ds(ci * CHUNK, CHUNK)],
                                      tbufs[tslot], tsems[tslot]),)

    def start(descs):
        for d in descs:
            d.start()

    def wait(descs):
        for d in descs:
            d.wait()

    def compute(slot, tslot):
        xb, tb = xbufs[slot], tbufs[tslot]

        def row(r, _):
            for j in range(NVEC):
                sl = pl.ds(j * LANES, LANES)
                sx = pl.ds(EMB + j * LANES, LANES)
                xb[r, sx] = xb[r, sx] + tb[r, sl]
            return 0

        lax.fori_loop(0, CHUNK, row, 0)

    def item_step(it, i, o, k, nb, epilogue):
        slot = o % NBUF
        s2 = (slot + LOOK) % NBUF
        if nb == 0:
            wait(tload_desc(0, k))
        wait(load_desc(slot, it))

        if not epilogue and o < LOOK:
            @pl.when(i >= 1)
            def _():
                wait(store_desc(s2, it - LOOK))
        else:
            wait(store_desc(s2, it - LOOK))

        compute(slot, k)
        start(store_desc(slot, it))
        if (not epilogue) or (it + LOOK < NITEM):
            start(load_desc(s2, it + LOOK))

    start(tload_desc(0, 0))
    for it in range(LOOK):
        start(load_desc(it, it))

    def round_step(i, _):
        for k in range(TSLOTS):
            ci = TSLOTS * i + k
            for nb in range(B_PER_W):
                o = B_PER_W * k + nb
                it = B_PER_W * ci + nb
                if nb == 0:
                    start(tload_desc(ci + 1, (k + 1) % TSLOTS))
                item_step(it, i, o, k, nb, epilogue=False)
        return 0

    lax.fori_loop(0, NROUND, round_step, 0)

    ci = NCH - 1
    for nb in range(B_PER_W):
        it = B_PER_W * ci + nb
        item_step(it, NROUND, it % (NBUF * 2), 0, nb, epilogue=True)
    for it in range(NITEM - LOOK, NITEM):
        wait(store_desc(it % NBUF, it))


def _sc_add(x, table):
    mesh = plsc.VectorSubcoreMesh(core_axis_name="c", subcore_axis_name="s")
    f = functools.partial(
        pl.kernel,
        mesh=mesh,
        out_type=jax.ShapeDtypeStruct((BATCH, NUM_POS, XD), jnp.float32),
        scratch_types=(
            [pltpu.VMEM((CHUNK, XD), jnp.float32) for _ in range(NBUF)]
            + [pltpu.VMEM((CHUNK, EMB), jnp.float32) for _ in range(TSLOTS)]
            + [pltpu.VMEM_SHARED((HALF_POS, EMB), jnp.float32)]
            + [pltpu.SemaphoreType.DMA for _ in range(2 * NBUF + TSLOTS)]),
    )(_sc_body)
    return f(x, table)


@jax.jit
def _run(x, table):
    return _sc_add(x, table)


def kernel(x, table):
    return _run(x, table)
